# baseline (speedup 1.0000x reference)
import functools
import jax
import jax.numpy as jnp
from jax import lax
from jax.experimental import pallas as pl
from jax.experimental.pallas import tpu as pltpu
from jax.experimental.pallas import tpu_sc as plsc

N = 10000
E = 320000
H = 128
DE = 16
DEPTH = 5
G_OUT = 200
PRED = 128
NG = 64

NC = 2
NS = 16
EPT = E // NS
K = 80
NCHUNK = EPT // K
NHALF = 5120
NACC = 5632
TRASH = NHALF
NPW = NACC // NS
AC = H + 16


def _lrelu(v):
    return jnp.maximum(v, 0.2 * v)


def _node0_body(x_ref, wi_ref, bi_ref, wc_ref, bc_ref, ai_ref, np_ref):
    ai = _lrelu(x_ref[...] @ wi_ref[...] + bi_ref[...])
    ai_ref[...] = ai
    np_ref[...] = ai @ wc_ref[...] + bc_ref[...]


def _node_body(a0_ref, wg_ref, ai_ref, wc_ref, bc_ref, np_ref, ah_ref):
    a0 = a0_ref[...]
    s = a0[:, :H]
    den = a0[:, H:H + 1]
    agg = jnp.where(den > 0, s / den, 0.0)
    attn_h = _lrelu(agg + wg_ref[...])
    ah = jax.nn.relu(attn_h + ai_ref[...])
    ah_ref[...] = ah
    np_ref[...] = ah @ wc_ref[...] + bc_ref[...]


def _edge_body(ea_ref, w_ref, b_ref, o_ref):
    o_ref[...] = ea_ref[...] @ w_ref[...] + b_ref[...]


def _readout_body(x_ref, a0_ref, wg_ref, ai_ref, goh_ref,
                  w1_ref, w2_ref, bo_ref, rw_ref, rb_ref, tw_ref, tb_ref,
                  o_ref, hs_ref, hm_ref):
    i = pl.program_id(0)

    @pl.when(i == 0)
    def _init():
        hs_ref[...] = jnp.zeros_like(hs_ref)
        hm_ref[...] = jnp.full_like(hm_ref, -jnp.inf)

    a0 = a0_ref[...]
    s = a0[:, :H]
    den = a0[:, H:H + 1]
    agg = jnp.where(den > 0, s / den, 0.0)
    attn_h = _lrelu(agg + wg_ref[...])
    ah = jax.nn.relu(attn_h + ai_ref[...])
    node_out = _lrelu(x_ref[...] @ w1_ref[...] + ah @ w2_ref[...] + bo_ref[...])
    wgt = jax.nn.sigmoid(node_out @ rw_ref[...] + rb_ref[...])
    goh = goh_ref[...]
    hs_ref[...] += goh.T @ (node_out * wgt)
    neg = jnp.float32(-jnp.inf)
    for g in range(NG):
        maskg = goh[:, g:g + 1] > 0
        cand = jnp.max(jnp.where(maskg, node_out, neg), axis=0)
        hm_ref[g, :] = jnp.maximum(hm_ref[g, :], cand)

    @pl.when(i == pl.num_programs(0) - 1)
    def _fin():
        hm = hm_ref[...]
        hm = jnp.where(jnp.isfinite(hm), hm, 0.0)
        hg = jnp.concatenate([hs_ref[...], hm], axis=1)
        o_ref[...] = hg @ tw_ref[...] + tb_ref[...]


_RT = 400
_ET = 2000


def _tc_node0(x, wi, bi, wc, bc):
    return pl.pallas_call(
        _node0_body,
        grid=(N // _RT,),
        in_specs=[
            pl.BlockSpec((_RT, H), lambda i: (i, 0)),
            pl.BlockSpec((H, H), lambda i: (0, 0)),
            pl.BlockSpec((1, H), lambda i: (0, 0)),
            pl.BlockSpec((H, 5 * H), lambda i: (0, 0)),
            pl.BlockSpec((1, 5 * H), lambda i: (0, 0)),
        ],
        out_specs=[
            pl.BlockSpec((_RT, H), lambda i: (i, 0)),
            pl.BlockSpec((_RT, 5 * H), lambda i: (i, 0)),
        ],
        out_shape=[
            jax.ShapeDtypeStruct((N, H), jnp.float32),
            jax.ShapeDtypeStruct((N, 5 * H), jnp.float32),
        ],
    )(x, wi, bi, wc, bc)


def _tc_node(a0, wg, ai, wc, bc):
    return pl.pallas_call(
        _node_body,
        grid=(N // _RT,),
        in_specs=[
            pl.BlockSpec((_RT, AC), lambda i: (i, 0)),
            pl.BlockSpec((_RT, H), lambda i: (i, 0)),
            pl.BlockSpec((_RT, H), lambda i: (i, 0)),
            pl.BlockSpec((H, 5 * H), lambda i: (0, 0)),
            pl.BlockSpec((1, 5 * H), lambda i: (0, 0)),
        ],
        out_specs=[
            pl.BlockSpec((_RT, 5 * H), lambda i: (i, 0)),
            pl.BlockSpec((_RT, H), lambda i: (i, 0)),
        ],
        out_shape=[
            jax.ShapeDtypeStruct((N, 5 * H), jnp.float32),
            jax.ShapeDtypeStruct((N, H), jnp.float32),
        ],
    )(a0, wg, ai, wc, bc)


def _tc_edge(ea, w, b):
    return pl.pallas_call(
        _edge_body,
        grid=(E // _ET,),
        in_specs=[
            pl.BlockSpec((_ET, DE), lambda i: (i, 0)),
            pl.BlockSpec((DE, 2 * H), lambda i: (0, 0)),
            pl.BlockSpec((1, 2 * H), lambda i: (0, 0)),
        ],
        out_specs=pl.BlockSpec((_ET, 2 * H), lambda i: (i, 0)),
        out_shape=jax.ShapeDtypeStruct((E, 2 * H), jnp.float32),
    )(ea, w, b)


def _tc_readout(x, a0, wg, ai, goh, w1, w2, bo, rw, rb, tw, tb):
    return pl.pallas_call(
        _readout_body,
        grid=(N // _RT,),
        in_specs=[
            pl.BlockSpec((_RT, H), lambda i: (i, 0)),
            pl.BlockSpec((_RT, AC), lambda i: (i, 0)),
            pl.BlockSpec((_RT, H), lambda i: (i, 0)),
            pl.BlockSpec((_RT, H), lambda i: (i, 0)),
            pl.BlockSpec((_RT, NG), lambda i: (i, 0)),
            pl.BlockSpec((H, G_OUT), lambda i: (0, 0)),
            pl.BlockSpec((H, G_OUT), lambda i: (0, 0)),
            pl.BlockSpec((1, G_OUT), lambda i: (0, 0)),
            pl.BlockSpec((G_OUT, 1), lambda i: (0, 0)),
            pl.BlockSpec((1, 1), lambda i: (0, 0)),
            pl.BlockSpec((2 * G_OUT, PRED), lambda i: (0, 0)),
            pl.BlockSpec((1, PRED), lambda i: (0, 0)),
        ],
        out_specs=pl.BlockSpec((NG, PRED), lambda i: (0, 0)),
        out_shape=jax.ShapeDtypeStruct((NG, PRED), jnp.float32),
        scratch_shapes=[
            pltpu.VMEM((NG, G_OUT), jnp.float32),
            pltpu.VMEM((NG, G_OUT), jnp.float32),
        ],
    )(x, a0, wg, ai, goh, w1, w2, bo, rw, rb, tw, tb)


def _sc_edge_body(src_tab, dst_tab, eproj, src_i, dst_i, wdot, zeros_hbm,
                  out_hbm, idx_s, idx_d, idx_d2, ra, rb, re, msgb, wdot_v,
                  spacc, sem):
    c = lax.axis_index("c")
    s = lax.axis_index("s")
    r0 = s * NPW
    pltpu.sync_copy(zeros_hbm.at[pl.ds(r0, NPW)], spacc.at[pl.ds(r0, NPW)])
    pltpu.sync_copy(wdot, wdot_v)
    plsc.subcore_barrier()

    base = s * EPT
    nbase = c * NHALF
    lanes = lax.iota(jnp.int32, 16)
    colH = jnp.full((16,), H, jnp.int32)

    def chunk_body(ci, carry):
        e0 = base + ci * K
        pltpu.sync_copy(src_i.at[pl.ds(e0, K)], idx_s)
        pltpu.sync_copy(dst_i.at[pl.ds(e0, K)], idx_d)
        cp1 = pltpu.async_copy(src_tab.at[idx_s], ra, sem)
        cp2 = pltpu.async_copy(dst_tab.at[idx_d], rb, sem)
        pltpu.sync_copy(eproj.at[pl.ds(e0, K)], re)
        for g in range(K // 16):
            gs = pl.ds(g * 16, 16)
            v = idx_d[gs] - nbase
            ok = (v >= 0) & (v < NHALF)
            idx_d2[gs] = jnp.where(ok, v, TRASH)
        cp1.wait()
        cp2.wait()

        def group_body(g, carry2):
            rows = lanes + g * 16

            def f_body(f, acc):
                for u in range(4):
                    fc = jnp.full((16,), f * 4 + u, jnp.int32)
                    va = plsc.load_gather(ra, [rows, fc])
                    vb = plsc.load_gather(rb, [rows, fc])
                    ve = plsc.load_gather(re, [rows, fc])
                    wv = plsc.load_gather(wdot_v, [fc])
                    acc = acc + _lrelu(va + vb + ve) * wv
                return acc

            sc = lax.fori_loop(0, H // 4, f_body,
                               jnp.zeros((16,), jnp.float32))
            ex = jnp.exp(sc)
            plsc.store_scatter(msgb, [rows, colH], ex)

            def m_body(f, carry3):
                for u in range(4):
                    fc = jnp.full((16,), f * 4 + u, jnp.int32)
                    fcm = jnp.full((16,), H + f * 4 + u, jnp.int32)
                    va = plsc.load_gather(ra, [rows, fcm])
                    vb = plsc.load_gather(rb, [rows, fcm])
                    ve = plsc.load_gather(re, [rows, fcm])
                    vm = ex * _lrelu(va + vb + ve)
                    plsc.store_scatter(msgb, [rows, fc], vm)
                return carry3

            lax.fori_loop(0, H // 4, m_body, 0)
            return carry2

        lax.fori_loop(0, K // 16, group_body, 0)
        pltpu.sync_copy(msgb, spacc.at[idx_d2], add=True)
        return carry

    lax.fori_loop(0, NCHUNK, chunk_body, 0)
    plsc.subcore_barrier()
    pltpu.sync_copy(spacc.at[pl.ds(r0, NPW)],
                    out_hbm.at[pl.ds(c * NACC + r0, NPW)])


_sc_edge = pl.kernel(
    _sc_edge_body,
    out_type=jax.ShapeDtypeStruct((NC * NACC, AC), jnp.float32),
    mesh=plsc.VectorSubcoreMesh(core_axis_name="c", subcore_axis_name="s"),
    compiler_params=pltpu.CompilerParams(use_tc_tiling_on_sc=False,
                                         needs_layout_passes=False),
    scratch_types=[
        pltpu.VMEM((K,), jnp.int32),
        pltpu.VMEM((K,), jnp.int32),
        pltpu.VMEM((K,), jnp.int32),
        pltpu.VMEM((K, 2 * H), jnp.float32),
        pltpu.VMEM((K, 2 * H), jnp.float32),
        pltpu.VMEM((K, 2 * H), jnp.float32),
        pltpu.VMEM((K, AC), jnp.float32),
        pltpu.VMEM((H,), jnp.float32),
        pltpu.VMEM_SHARED((NACC, AC), jnp.float32),
        pltpu.SemaphoreType.DMA,
    ],
)


def kernel(x, edge_index, edge_attr, node_graph_ids,
           atom_inp_w, atom_inp_b, attn_src_w, attn_src_b,
           attn_dst_w, attn_dst_b, attn_edg_w, attn_edg_b,
           attn_dot_w, attn_dot_b, msg_src_w, msg_src_b,
           msg_dst_w, msg_dst_b, msg_edg_w, msg_edg_b,
           wgt_n_w, wgt_n_b, atom_out_w, atom_out_b,
           readout_w, readout_b, transform_w, transform_b):
    src = edge_index[0].astype(jnp.int32)
    dst = edge_index[1].astype(jnp.int32)
    zeros_acc = jnp.zeros((NACC, AC), jnp.float32)

    wcs = [jnp.concatenate([attn_src_w[l], msg_src_w[l], attn_dst_w[l],
                            msg_dst_w[l], wgt_n_w[l]], axis=1)
           for l in range(DEPTH)]
    bcs = [jnp.concatenate([attn_src_b[l], msg_src_b[l], attn_dst_b[l],
                            msg_dst_b[l], wgt_n_b[l]])[None, :]
           for l in range(DEPTH)]
    wes = [jnp.concatenate([attn_edg_w[l], msg_edg_w[l]], axis=1)
           for l in range(DEPTH)]
    bes = [jnp.concatenate([attn_edg_b[l], msg_edg_b[l]])[None, :]
           for l in range(DEPTH)]

    atom_input, nproj = _tc_node0(x, atom_inp_w, atom_inp_b[None, :],
                                  wcs[0], bcs[0])

    agg = None
    for l in range(DEPTH):
        eproj = _tc_edge(edge_attr, wes[l], bes[l])
        src_tab = nproj[:, :2 * H]
        dst_tab = nproj[:, 2 * H:4 * H]
        wgtn = nproj[:, 4 * H:]
        acc2 = _sc_edge(src_tab, dst_tab, eproj, src, dst,
                        attn_dot_w[l][:, 0], zeros_acc)
        agg = jnp.concatenate([acc2[:NHALF], acc2[NACC:NACC + N - NHALF]],
                              axis=0)
        if l < DEPTH - 1:
            nproj, _ = _tc_node(agg, wgtn, atom_input,
                                wcs[l + 1], bcs[l + 1])

    goh = (node_graph_ids[:, None] == jnp.arange(NG)[None, :]).astype(jnp.float32)
    out = _tc_readout(x, agg, wgtn, atom_input, goh,
                      atom_out_w[:128], atom_out_w[128:], atom_out_b[None, :],
                      readout_w, readout_b[None, :],
                      transform_w, transform_b[None, :])
    return out


# speedup vs baseline: 1.0265x; 1.0265x over previous
import functools
import jax
import jax.numpy as jnp
from jax import lax
from jax.experimental import pallas as pl
from jax.experimental.pallas import tpu as pltpu
from jax.experimental.pallas import tpu_sc as plsc

N = 10000
E = 320000
H = 128
DE = 16
DEPTH = 5
G_OUT = 200
PRED = 128
NG = 64

NC = 2
NS = 16
EPT = E // NS
K = 80
NCHUNK = EPT // K
NHALF = 5120
NACC = 5632
TRASH = NHALF
NPW = NACC // NS
AC = H + 16


def _lrelu(v):
    return jnp.maximum(v, 0.2 * v)


def _node0_body(x_ref, wi_ref, bi_ref, wc_ref, bc_ref, ai_ref, np_ref):
    ai = _lrelu(x_ref[...] @ wi_ref[...] + bi_ref[...])
    ai_ref[...] = ai
    np_ref[...] = ai @ wc_ref[...] + bc_ref[...]


def _node_body(a0_ref, wg_ref, ai_ref, wc_ref, bc_ref, np_ref, ah_ref):
    a0 = a0_ref[...]
    s = a0[:, :H]
    den = a0[:, H:H + 1]
    agg = jnp.where(den > 0, s / den, 0.0)
    attn_h = _lrelu(agg + wg_ref[...])
    ah = jax.nn.relu(attn_h + ai_ref[...])
    ah_ref[...] = ah
    np_ref[...] = ah @ wc_ref[...] + bc_ref[...]


def _edge_body(ea_ref, w_ref, b_ref, o_ref):
    o_ref[...] = ea_ref[...] @ w_ref[...] + b_ref[...]


def _readout_body(x_ref, a0_ref, wg_ref, ai_ref, goh_ref,
                  w1_ref, w2_ref, bo_ref, rw_ref, rb_ref, tw_ref, tb_ref,
                  o_ref, hs_ref, hm_ref):
    i = pl.program_id(0)

    @pl.when(i == 0)
    def _init():
        hs_ref[...] = jnp.zeros_like(hs_ref)
        hm_ref[...] = jnp.full_like(hm_ref, -jnp.inf)

    a0 = a0_ref[...]
    s = a0[:, :H]
    den = a0[:, H:H + 1]
    agg = jnp.where(den > 0, s / den, 0.0)
    attn_h = _lrelu(agg + wg_ref[...])
    ah = jax.nn.relu(attn_h + ai_ref[...])
    node_out = _lrelu(x_ref[...] @ w1_ref[...] + ah @ w2_ref[...] + bo_ref[...])
    wgt = jax.nn.sigmoid(node_out @ rw_ref[...] + rb_ref[...])
    goh = goh_ref[...]
    hs_ref[...] += goh.T @ (node_out * wgt)
    neg = jnp.float32(-jnp.inf)
    for g in range(NG):
        maskg = goh[:, g:g + 1] > 0
        cand = jnp.max(jnp.where(maskg, node_out, neg), axis=0)
        hm_ref[g, :] = jnp.maximum(hm_ref[g, :], cand)

    @pl.when(i == pl.num_programs(0) - 1)
    def _fin():
        hm = hm_ref[...]
        hm = jnp.where(jnp.isfinite(hm), hm, 0.0)
        hg = jnp.concatenate([hs_ref[...], hm], axis=1)
        o_ref[...] = hg @ tw_ref[...] + tb_ref[...]


_RT = 400
_ET = 2000


def _tc_node0(x, wi, bi, wc, bc):
    return pl.pallas_call(
        _node0_body,
        grid=(N // _RT,),
        in_specs=[
            pl.BlockSpec((_RT, H), lambda i: (i, 0)),
            pl.BlockSpec((H, H), lambda i: (0, 0)),
            pl.BlockSpec((1, H), lambda i: (0, 0)),
            pl.BlockSpec((H, 5 * H), lambda i: (0, 0)),
            pl.BlockSpec((1, 5 * H), lambda i: (0, 0)),
        ],
        out_specs=[
            pl.BlockSpec((_RT, H), lambda i: (i, 0)),
            pl.BlockSpec((_RT, 5 * H), lambda i: (i, 0)),
        ],
        out_shape=[
            jax.ShapeDtypeStruct((N, H), jnp.float32),
            jax.ShapeDtypeStruct((N, 5 * H), jnp.float32),
        ],
    )(x, wi, bi, wc, bc)


def _tc_node(a0, wg, ai, wc, bc):
    return pl.pallas_call(
        _node_body,
        grid=(N // _RT,),
        in_specs=[
            pl.BlockSpec((_RT, AC), lambda i: (i, 0)),
            pl.BlockSpec((_RT, H), lambda i: (i, 0)),
            pl.BlockSpec((_RT, H), lambda i: (i, 0)),
            pl.BlockSpec((H, 5 * H), lambda i: (0, 0)),
            pl.BlockSpec((1, 5 * H), lambda i: (0, 0)),
        ],
        out_specs=[
            pl.BlockSpec((_RT, 5 * H), lambda i: (i, 0)),
            pl.BlockSpec((_RT, H), lambda i: (i, 0)),
        ],
        out_shape=[
            jax.ShapeDtypeStruct((N, 5 * H), jnp.float32),
            jax.ShapeDtypeStruct((N, H), jnp.float32),
        ],
    )(a0, wg, ai, wc, bc)


def _tc_edge(ea, w, b):
    return pl.pallas_call(
        _edge_body,
        grid=(E // _ET,),
        in_specs=[
            pl.BlockSpec((_ET, DE), lambda i: (i, 0)),
            pl.BlockSpec((DE, 2 * H), lambda i: (0, 0)),
            pl.BlockSpec((1, 2 * H), lambda i: (0, 0)),
        ],
        out_specs=pl.BlockSpec((_ET, 2 * H), lambda i: (i, 0)),
        out_shape=jax.ShapeDtypeStruct((E, 2 * H), jnp.float32),
    )(ea, w, b)


def _tc_readout(x, a0, wg, ai, goh, w1, w2, bo, rw, rb, tw, tb):
    return pl.pallas_call(
        _readout_body,
        grid=(N // _RT,),
        in_specs=[
            pl.BlockSpec((_RT, H), lambda i: (i, 0)),
            pl.BlockSpec((_RT, AC), lambda i: (i, 0)),
            pl.BlockSpec((_RT, H), lambda i: (i, 0)),
            pl.BlockSpec((_RT, H), lambda i: (i, 0)),
            pl.BlockSpec((_RT, NG), lambda i: (i, 0)),
            pl.BlockSpec((H, G_OUT), lambda i: (0, 0)),
            pl.BlockSpec((H, G_OUT), lambda i: (0, 0)),
            pl.BlockSpec((1, G_OUT), lambda i: (0, 0)),
            pl.BlockSpec((G_OUT, 1), lambda i: (0, 0)),
            pl.BlockSpec((1, 1), lambda i: (0, 0)),
            pl.BlockSpec((2 * G_OUT, PRED), lambda i: (0, 0)),
            pl.BlockSpec((1, PRED), lambda i: (0, 0)),
        ],
        out_specs=pl.BlockSpec((NG, PRED), lambda i: (0, 0)),
        out_shape=jax.ShapeDtypeStruct((NG, PRED), jnp.float32),
        scratch_shapes=[
            pltpu.VMEM((NG, G_OUT), jnp.float32),
            pltpu.VMEM((NG, G_OUT), jnp.float32),
        ],
    )(x, a0, wg, ai, goh, w1, w2, bo, rw, rb, tw, tb)


def _sc_edge_body(src_tab, dst_tab, eproj, src_i, dst_i, wdot, zeros_hbm,
                  out_hbm, idx_s, idx_d, idx_d2, ra, rb, re, msgb, wdot_v,
                  spacc, sem):
    c = lax.axis_index("c")
    s = lax.axis_index("s")
    r0 = s * NPW
    pltpu.sync_copy(zeros_hbm.at[pl.ds(r0, NPW)], spacc.at[pl.ds(r0, NPW)])
    pltpu.sync_copy(wdot, wdot_v)
    plsc.subcore_barrier()

    base = s * EPT
    nbase = c * NHALF
    lanes = lax.iota(jnp.int32, 16)
    colH = jnp.full((16,), H, jnp.int32)

    def chunk_body(ci, carry):
        e0 = base + ci * K
        pltpu.sync_copy(src_i.at[pl.ds(e0, K)], idx_s)
        pltpu.sync_copy(dst_i.at[pl.ds(e0, K)], idx_d)
        cp1 = pltpu.async_copy(src_tab.at[idx_s], ra, sem)
        cp2 = pltpu.async_copy(dst_tab.at[idx_d], rb, sem)
        pltpu.sync_copy(eproj.at[pl.ds(e0, K)], re)
        for g in range(K // 16):
            gs = pl.ds(g * 16, 16)
            v = idx_d[gs] - nbase
            ok = (v >= 0) & (v < NHALF)
            idx_d2[gs] = jnp.where(ok, v, TRASH)
        cp1.wait()
        cp2.wait()

        def group_body(g, carry2):
            rows = lanes + g * 16

            def f_body(f, accs):
                acc_a, acc_b = accs
                for u in range(8):
                    fc = jnp.full((16,), f * 8 + u, jnp.int32)
                    va = plsc.load_gather(ra, [rows, fc])
                    vb = plsc.load_gather(rb, [rows, fc])
                    ve = plsc.load_gather(re, [rows, fc])
                    wv = plsc.load_gather(wdot_v, [fc])
                    t = _lrelu(va + vb + ve) * wv
                    if u % 2 == 0:
                        acc_a = acc_a + t
                    else:
                        acc_b = acc_b + t
                return acc_a, acc_b

            z16 = jnp.zeros((16,), jnp.float32)
            acc_a, acc_b = lax.fori_loop(0, H // 8, f_body, (z16, z16))
            ex = jnp.exp(acc_a + acc_b)
            plsc.store_scatter(msgb, [rows, colH], ex)

            def m_body(f, carry3):
                for u in range(8):
                    fc = jnp.full((16,), f * 8 + u, jnp.int32)
                    fcm = jnp.full((16,), H + f * 8 + u, jnp.int32)
                    va = plsc.load_gather(ra, [rows, fcm])
                    vb = plsc.load_gather(rb, [rows, fcm])
                    ve = plsc.load_gather(re, [rows, fcm])
                    vm = ex * _lrelu(va + vb + ve)
                    plsc.store_scatter(msgb, [rows, fc], vm)
                return carry3

            lax.fori_loop(0, H // 8, m_body, 0)
            return carry2

        lax.fori_loop(0, K // 16, group_body, 0)
        pltpu.sync_copy(msgb, spacc.at[idx_d2], add=True)
        return carry

    lax.fori_loop(0, NCHUNK, chunk_body, 0)
    plsc.subcore_barrier()
    pltpu.sync_copy(spacc.at[pl.ds(r0, NPW)],
                    out_hbm.at[pl.ds(c * NACC + r0, NPW)])


_sc_edge = pl.kernel(
    _sc_edge_body,
    out_type=jax.ShapeDtypeStruct((NC * NACC, AC), jnp.float32),
    mesh=plsc.VectorSubcoreMesh(core_axis_name="c", subcore_axis_name="s"),
    compiler_params=pltpu.CompilerParams(use_tc_tiling_on_sc=False,
                                         needs_layout_passes=False),
    scratch_types=[
        pltpu.VMEM((K,), jnp.int32),
        pltpu.VMEM((K,), jnp.int32),
        pltpu.VMEM((K,), jnp.int32),
        pltpu.VMEM((K, 2 * H), jnp.float32),
        pltpu.VMEM((K, 2 * H), jnp.float32),
        pltpu.VMEM((K, 2 * H), jnp.float32),
        pltpu.VMEM((K, AC), jnp.float32),
        pltpu.VMEM((H,), jnp.float32),
        pltpu.VMEM_SHARED((NACC, AC), jnp.float32),
        pltpu.SemaphoreType.DMA,
    ],
)


def kernel(x, edge_index, edge_attr, node_graph_ids,
           atom_inp_w, atom_inp_b, attn_src_w, attn_src_b,
           attn_dst_w, attn_dst_b, attn_edg_w, attn_edg_b,
           attn_dot_w, attn_dot_b, msg_src_w, msg_src_b,
           msg_dst_w, msg_dst_b, msg_edg_w, msg_edg_b,
           wgt_n_w, wgt_n_b, atom_out_w, atom_out_b,
           readout_w, readout_b, transform_w, transform_b):
    src = edge_index[0].astype(jnp.int32)
    dst = edge_index[1].astype(jnp.int32)
    zeros_acc = jnp.zeros((NACC, AC), jnp.float32)

    wcs = [jnp.concatenate([attn_src_w[l], msg_src_w[l], attn_dst_w[l],
                            msg_dst_w[l], wgt_n_w[l]], axis=1)
           for l in range(DEPTH)]
    bcs = [jnp.concatenate([attn_src_b[l], msg_src_b[l], attn_dst_b[l],
                            msg_dst_b[l], wgt_n_b[l]])[None, :]
           for l in range(DEPTH)]
    wes = [jnp.concatenate([attn_edg_w[l], msg_edg_w[l]], axis=1)
           for l in range(DEPTH)]
    bes = [jnp.concatenate([attn_edg_b[l], msg_edg_b[l]])[None, :]
           for l in range(DEPTH)]

    atom_input, nproj = _tc_node0(x, atom_inp_w, atom_inp_b[None, :],
                                  wcs[0], bcs[0])

    agg = None
    for l in range(DEPTH):
        eproj = _tc_edge(edge_attr, wes[l], bes[l])
        src_tab = nproj[:, :2 * H]
        dst_tab = nproj[:, 2 * H:4 * H]
        wgtn = nproj[:, 4 * H:]
        acc2 = _sc_edge(src_tab, dst_tab, eproj, src, dst,
                        attn_dot_w[l][:, 0], zeros_acc)
        agg = jnp.concatenate([acc2[:NHALF], acc2[NACC:NACC + N - NHALF]],
                              axis=0)
        if l < DEPTH - 1:
            nproj, _ = _tc_node(agg, wgtn, atom_input,
                                wcs[l + 1], bcs[l + 1])

    goh = (node_graph_ids[:, None] == jnp.arange(NG)[None, :]).astype(jnp.float32)
    out = _tc_readout(x, agg, wgtn, atom_input, goh,
                      atom_out_w[:128], atom_out_w[128:], atom_out_b[None, :],
                      readout_w, readout_b[None, :],
                      transform_w, transform_b[None, :])
    return out
